# baseline (speedup 1.0000x reference)
import jax
import jax.numpy as jnp
from jax.experimental import pallas as pl

_NV = 8
_GB = 8


def _fast_body(x_ref, o_ref, t_ref):
    block = x_ref[...]
    gb, nv, n = block.shape
    rowmax = jnp.max(block, axis=2)
    m = jnp.max(rowmax, axis=1, keepdims=True)
    ismax = rowmax == m
    counts = jnp.sum(ismax.astype(jnp.int32), axis=1, keepdims=True)
    viota = jax.lax.broadcasted_iota(jnp.int32, (gb, nv), 1)
    votes = jnp.min(jnp.where(ismax, viota, jnp.int32(nv)), axis=1)
    t_ref[...] = counts.reshape(gb, 1, 1)
    for g in range(gb):
        o_ref[g, 0, :] = x_ref[g, votes[g], :]


def _exact_body(x_ref, o_ref):
    for g in range(_GB):
        data = x_ref[g]
        nv, n = data.shape
        m = jnp.max(data)
        vgrid = jax.lax.broadcasted_iota(jnp.int32, (nv, n), 0)
        fgrid = jax.lax.broadcasted_iota(jnp.int32, (nv, n), 1)
        keys = jnp.where(data == m, fgrid * nv + vgrid, jnp.int32(2**31 - 1))
        vote = jnp.min(keys) % nv
        o_ref[g, 0, :] = x_ref[g, vote, :]


def _make_fast(b, n, interpret=False):
    return pl.pallas_call(
        _fast_body,
        grid=(b // _GB,),
        in_specs=[pl.BlockSpec((_GB, _NV, n), lambda g: (g, 0, 0))],
        out_specs=[
            pl.BlockSpec((_GB, 1, n), lambda g: (g, 0, 0)),
            pl.BlockSpec((_GB, 1, 1), lambda g: (g, 0, 0)),
        ],
        out_shape=[
            jax.ShapeDtypeStruct((b, 1, n), jnp.float32),
            jax.ShapeDtypeStruct((b, 1, 1), jnp.int32),
        ],
        interpret=interpret,
    )


def _make_exact(b, n, interpret=False):
    return pl.pallas_call(
        _exact_body,
        grid=(b // _GB,),
        in_specs=[pl.BlockSpec((_GB, _NV, n), lambda g: (g, 0, 0))],
        out_specs=pl.BlockSpec((_GB, 1, n), lambda g: (g, 0, 0)),
        out_shape=jax.ShapeDtypeStruct((b, 1, n), jnp.float32),
        interpret=interpret,
    )


def _run(x, interpret=False):
    b = x.shape[0] // _NV
    xr = x.reshape(b, _NV, -1)
    n = xr.shape[-1]
    out, counts = _make_fast(b, n, interpret)(xr)
    tie = jnp.any(counts > 1)
    out = jax.lax.cond(
        tie,
        lambda: _make_exact(b, n, interpret)(xr),
        lambda: out,
    )
    return out.reshape(b, n)


def kernel(x):
    return _run(x)


# speedup vs baseline: 2.1839x; 2.1839x over previous
import jax
import jax.numpy as jnp
from jax.experimental import pallas as pl

_NV = 8
_GB = 8


def _fast_body(x_ref, o_ref):
    block = x_ref[...]
    gb, nv, n = block.shape
    rowmax = jnp.max(block, axis=2)
    m = jnp.max(rowmax, axis=1, keepdims=True)
    ismax = rowmax == m
    counts = jnp.sum(ismax.astype(jnp.int32), axis=1)
    viota = jax.lax.broadcasted_iota(jnp.int32, (gb, nv), 1)
    votes_fast = jnp.min(jnp.where(ismax, viota, jnp.int32(nv)), axis=1)

    def _tie_vote(g):
        def _inner():
            vgrid = jax.lax.broadcasted_iota(jnp.int32, (nv, n), 0)
            fgrid = jax.lax.broadcasted_iota(jnp.int32, (nv, n), 1)
            keys = jnp.where(block[g] == m[g, 0], fgrid * nv + vgrid,
                             jnp.int32(2**31 - 1))
            return jnp.min(keys) % nv
        return _inner

    for g in range(gb):
        vote = jax.lax.cond(counts[g] > 1, _tie_vote(g),
                            lambda vf=votes_fast[g]: vf)
        o_ref[g, 0, :] = x_ref[g, vote, :]


def _make_fast(b, n, interpret=False):
    return pl.pallas_call(
        _fast_body,
        grid=(b // _GB,),
        in_specs=[pl.BlockSpec((_GB, _NV, n), lambda g: (g, 0, 0))],
        out_specs=pl.BlockSpec((_GB, 1, n), lambda g: (g, 0, 0)),
        out_shape=jax.ShapeDtypeStruct((b, 1, n), jnp.float32),
        interpret=interpret,
    )


def _run(x, interpret=False):
    b = x.shape[0] // _NV
    xr = x.reshape(b, _NV, -1)
    n = xr.shape[-1]
    out = _make_fast(b, n, interpret)(xr)
    return out.reshape(b, n)


def kernel(x):
    return _run(x)
